# baseline (speedup 1.0000x reference)
import jax
import jax.numpy as jnp
from jax import lax
from jax.experimental import pallas as pl
from jax.experimental.pallas import tpu as pltpu
from jax.experimental.pallas import tpu_sc as plsc

N = 10000
D = 256
H = 128
E = 160000
NC, NS, L = 2, 16, 16
NW = NC * NS
K = 128
EPT = 5120
EPAD = EPT * NW
NCHUNK = EPT // K
AC = EPAD // (NS * K)
HB = AC // 2
NNP = 10240
RPT = NNP // NS
RB = 128
DEG_PAD = 10016
BN = 1280
GRID = NNP // BN

_sc_mesh = plsc.VectorSubcoreMesh(
    core_axis_name="c", subcore_axis_name="s", num_cores=NC, num_subcores=NS
)


def _deg_body(dst_hbm, out_hbm, dst_v, ones_v, zeros_v, shared):
    c = lax.axis_index("c")
    s = lax.axis_index("s")
    wid = s * NC + c
    pltpu.sync_copy(dst_hbm.at[pl.ds(wid * NCHUNK, NCHUNK)], dst_v)

    one16 = jnp.ones((L,), jnp.float32)

    def fbody(i, carry):
        ones_v[pl.ds(i * L, L)] = one16
        return carry

    lax.fori_loop(0, K // L, fbody, 0)

    @pl.when(s == 0)
    def _():
        zero16 = jnp.zeros((L,), jnp.float32)

        def zbody(i, carry):
            zeros_v[pl.ds(i * L, L)] = zero16
            return carry

        lax.fori_loop(0, DEG_PAD // L, zbody, 0)
        pltpu.sync_copy(zeros_v, shared)

    plsc.subcore_barrier()

    def ebody(j, carry):
        pltpu.sync_copy(ones_v, shared.at[dst_v.at[j]], add=True)
        return carry

    lax.fori_loop(0, NCHUNK, ebody, 0)

    plsc.subcore_barrier()

    @pl.when(s == 0)
    def _():
        pltpu.sync_copy(shared, zeros_v)
        pltpu.sync_copy(zeros_v.at[pl.ds(0, N)], out_hbm.at[pl.ds(c * N, N)])


_deg_kernel = pl.kernel(
    _deg_body,
    out_type=jax.ShapeDtypeStruct((2 * N,), jnp.float32),
    mesh=_sc_mesh,
    scratch_types=[
        pltpu.VMEM((NCHUNK, K), jnp.int32),
        pltpu.VMEM((K,), jnp.float32),
        pltpu.VMEM((DEG_PAD,), jnp.float32),
        pltpu.VMEM_SHARED((DEG_PAD,), jnp.float32),
    ],
)


def _agg_body(src_hbm, dst_hbm, y_hbm, o_hbm, src_v, dst_v, rows0, rows1,
              acc, sem0, sem1):
    c = lax.axis_index("c")
    s = lax.axis_index("s")
    base = c * NNP

    def ibody(j, carry):
        off = s * RPT + j * RB
        pltpu.sync_copy(y_hbm.at[pl.ds(base + off, RB)], rows0)
        pltpu.sync_copy(rows0, acc.at[pl.ds(off, RB)])
        return carry

    lax.fori_loop(0, RPT // RB, ibody, 0)

    plsc.subcore_barrier()

    def gather(jj, buf, sem):
        pltpu.async_copy(y_hbm.at[src_v.at[2 * jj]], buf.at[pl.ds(0, K // 2)], sem)
        pltpu.async_copy(y_hbm.at[src_v.at[2 * jj + 1]], buf.at[pl.ds(K // 2, K // 2)], sem)

    def gwait(buf, sem):
        pltpu.make_async_copy(y_hbm.at[src_v.at[0]], buf.at[pl.ds(0, K // 2)], sem).wait()
        pltpu.make_async_copy(y_hbm.at[src_v.at[0]], buf.at[pl.ds(K // 2, K // 2)], sem).wait()

    def batch(b, carry):
        pltpu.sync_copy(
            src_hbm.at[pl.ds(2 * (c * (EPAD // K) + s * AC + b * HB), 2 * HB)],
            src_v)
        pltpu.sync_copy(dst_hbm.at[pl.ds(s * AC + b * HB, HB)], dst_v)
        gather(0, rows0, sem0)

        def ebody(t, carry2):
            j0 = 2 * t
            j1 = j0 + 1
            gather(j1, rows1, sem1)
            gwait(rows0, sem0)
            pltpu.sync_copy(rows0, acc.at[dst_v.at[j0]], add=True)

            @pl.when(j0 + 2 < HB)
            def _():
                gather(j0 + 2, rows0, sem0)

            gwait(rows1, sem1)
            pltpu.sync_copy(rows1, acc.at[dst_v.at[j1]], add=True)
            return carry2

        lax.fori_loop(0, HB // 2, ebody, 0)
        return carry

    lax.fori_loop(0, 2, batch, 0)

    plsc.subcore_barrier()

    def obody(j, carry):
        off = s * RPT + j * RB
        pltpu.sync_copy(acc.at[pl.ds(off, RB)], rows0)
        pltpu.sync_copy(rows0, o_hbm.at[pl.ds(base + off, RB)])
        return carry

    lax.fori_loop(0, RPT // RB, obody, 0)


_agg_kernel = pl.kernel(
    _agg_body,
    out_type=jax.ShapeDtypeStruct((2 * NNP, H), jnp.float32),
    mesh=_sc_mesh,
    scratch_types=[
        pltpu.VMEM((2 * HB, K // 2), jnp.int32),
        pltpu.VMEM((HB, K), jnp.int32),
        pltpu.VMEM((K, H), jnp.float32),
        pltpu.VMEM((K, H), jnp.float32),
        pltpu.VMEM_SHARED((NNP, H), jnp.float32),
        pltpu.SemaphoreType.DMA,
        pltpu.SemaphoreType.DMA,
    ],
)


def _dinv_of(degp_ref):
    deg = jnp.sum(degp_ref[...], axis=1) + 1.0
    return lax.rsqrt(deg)


def _tc_in_body(x_ref, w1_ref, degp_ref, y0_ref, y1_ref):
    dinv = _dinv_of(degp_ref)
    xw = jnp.dot(x_ref[...], w1_ref[...], preferred_element_type=jnp.float32)
    y = xw * dinv[:, None]
    y0_ref[...] = y[:, :H]
    y1_ref[...] = y[:, H:]


def _tc_mid_body(a0_ref, a1_ref, degp_ref, b1_ref, w2_ref,
                 h1_ref, y0_ref, y1_ref):
    dinv = _dinv_of(degp_ref)
    agg = jnp.concatenate([a0_ref[...], a1_ref[...]], axis=1)
    h1 = jnp.maximum(agg * dinv[:, None] + b1_ref[0, :], 0.0)
    h1_ref[...] = h1
    y2 = jnp.dot(h1, w2_ref[...], preferred_element_type=jnp.float32)
    y2 = y2 * dinv[:, None]
    y0_ref[...] = y2[:, :H]
    y1_ref[...] = y2[:, H:]


def _tc_out_body(a0_ref, a1_ref, degp_ref, b2_ref, h1_ref, wr_ref, br_ref,
                 out_ref):
    dinv = _dinv_of(degp_ref)
    agg = jnp.concatenate([a0_ref[...], a1_ref[...]], axis=1)
    h2 = jnp.maximum(agg * dinv[:, None] + b2_ref[0, :], 0.0)
    logits = (
        jnp.dot(h1_ref[...], wr_ref[:D, :], preferred_element_type=jnp.float32)
        + jnp.dot(h2, wr_ref[D:, :], preferred_element_type=jnp.float32)
        + br_ref[0, :]
    )
    m = jnp.max(logits, axis=1, keepdims=True)
    z = logits - m
    lse = jnp.log(jnp.sum(jnp.exp(z), axis=1, keepdims=True))
    out_ref[...] = z - lse


def _node_blk(shape_minor):
    return pl.BlockSpec((BN,) + shape_minor, lambda i: (i,) + (0,) * len(shape_minor))


def _slab_blk(half):
    return pl.BlockSpec((BN, H), lambda i, _h=half: (_h * GRID + i, 0))


_degp_blk = pl.BlockSpec((BN, 2), lambda i: (i, 0))


def _full_blk(shape):
    return pl.BlockSpec(shape, lambda i: (0,) * len(shape))


_tc_in = pl.pallas_call(
    _tc_in_body,
    grid=(GRID,),
    in_specs=[_node_blk((D,)), _full_blk((D, D)), _degp_blk],
    out_specs=[_node_blk((H,)), _node_blk((H,))],
    out_shape=(
        jax.ShapeDtypeStruct((NNP, H), jnp.float32),
        jax.ShapeDtypeStruct((NNP, H), jnp.float32),
    ),
)

_tc_mid = pl.pallas_call(
    _tc_mid_body,
    grid=(GRID,),
    in_specs=[_slab_blk(0), _slab_blk(1), _degp_blk,
              _full_blk((1, D)), _full_blk((D, D))],
    out_specs=[_node_blk((D,)), _node_blk((H,)), _node_blk((H,))],
    out_shape=(
        jax.ShapeDtypeStruct((N, D), jnp.float32),
        jax.ShapeDtypeStruct((NNP, H), jnp.float32),
        jax.ShapeDtypeStruct((NNP, H), jnp.float32),
    ),
)

_tc_out = pl.pallas_call(
    _tc_out_body,
    grid=(GRID,),
    in_specs=[_slab_blk(0), _slab_blk(1), _degp_blk,
              _full_blk((1, D)), _node_blk((D,)), _full_blk((2 * D, 40)),
              _full_blk((1, 40))],
    out_specs=_node_blk((40,)),
    out_shape=jax.ShapeDtypeStruct((N, 40), jnp.float32),
)


def kernel(x, edge_index, W1, b1, W2, b2, Wr, br):
    src = edge_index[0].astype(jnp.int32)
    dst = edge_index[1].astype(jnp.int32)
    npad = EPAD - E
    src_p = jnp.concatenate([src, jnp.zeros((npad,), jnp.int32)])
    dst_p = jnp.concatenate([dst, jnp.full((npad,), N, jnp.int32)])
    src2d = src_p.reshape(EPAD // K, K)
    dst2d = dst_p.reshape(EPAD // K, K)
    gsrc2d = jnp.concatenate([src2d, src2d + NNP], axis=0).reshape(-1, K // 2)

    degp = _deg_kernel(dst2d).reshape(2, N).T

    y0, y1 = _tc_in(x, W1, degp)
    a = _agg_kernel(gsrc2d, dst2d, jnp.concatenate([y0, y1], axis=0))
    h1, z0, z1 = _tc_mid(a, a, degp, b1.reshape(1, D), W2)
    g = _agg_kernel(gsrc2d, dst2d, jnp.concatenate([z0, z1], axis=0))
    return _tc_out(g, g, degp, b2.reshape(1, D), h1, Wr, br.reshape(1, 40))


# speedup vs baseline: 2.2069x; 2.2069x over previous
import jax
import jax.numpy as jnp
from jax import lax
from jax.experimental import pallas as pl
from jax.experimental.pallas import tpu as pltpu
from jax.experimental.pallas import tpu_sc as plsc

N = 10000
D = 256
H = 128
E = 160000
NC, NS, L = 2, 16, 16
NW = NC * NS
K = 125
NROW = E // K
NCHUNK = E // (NW * K)
AC = E // (NS * K)
HB = AC // 2
NNP = 10240
RPT = NNP // NS
RB = 128
DEG_PAD = 10016
BN = 1280
GRID = NNP // BN

_sc_mesh = plsc.VectorSubcoreMesh(
    core_axis_name="c", subcore_axis_name="s", num_cores=NC, num_subcores=NS
)


def _deg_body(dst_hbm, out_hbm, dst_v, ones_v, zeros_v, shared):
    c = lax.axis_index("c")
    s = lax.axis_index("s")
    wid = s * NC + c
    pltpu.sync_copy(dst_hbm.at[pl.ds(wid * NCHUNK, NCHUNK)], dst_v)

    one16 = jnp.ones((L,), jnp.float32)

    def fbody(i, carry):
        ones_v[pl.ds(i * L, L)] = one16
        return carry

    lax.fori_loop(0, 8, fbody, 0)

    @pl.when(s == 0)
    def _():
        zero16 = jnp.zeros((L,), jnp.float32)

        def zbody(i, carry):
            zeros_v[pl.ds(i * L, L)] = zero16
            return carry

        lax.fori_loop(0, DEG_PAD // L, zbody, 0)
        pltpu.sync_copy(zeros_v, shared)

    plsc.subcore_barrier()

    def ebody(j, carry):
        pltpu.sync_copy(ones_v.at[pl.ds(0, K)], shared.at[dst_v.at[j]], add=True)
        return carry

    lax.fori_loop(0, NCHUNK, ebody, 0)

    plsc.subcore_barrier()

    @pl.when(s == 0)
    def _():
        pltpu.sync_copy(shared, zeros_v)
        pltpu.sync_copy(zeros_v.at[pl.ds(0, N)], out_hbm.at[pl.ds(c * N, N)])


_deg_kernel = pl.kernel(
    _deg_body,
    out_type=jax.ShapeDtypeStruct((2 * N,), jnp.float32),
    mesh=_sc_mesh,
    scratch_types=[
        pltpu.VMEM((NCHUNK, K), jnp.int32),
        pltpu.VMEM((RB,), jnp.float32),
        pltpu.VMEM((DEG_PAD,), jnp.float32),
        pltpu.VMEM_SHARED((DEG_PAD,), jnp.float32),
    ],
)


def _agg_body(src_hbm, dst_hbm, y_hbm, o_hbm, src_v, dst_v, rows0, rows1,
              acc, sem0, sem1):
    c = lax.axis_index("c")
    s = lax.axis_index("s")
    base = c * NNP

    def ibody(j, carry):
        off = s * RPT + j * RB
        pltpu.sync_copy(y_hbm.at[pl.ds(base + off, RB)], rows0)
        pltpu.sync_copy(rows0, acc.at[pl.ds(off, RB)])
        return carry

    lax.fori_loop(0, RPT // RB, ibody, 0)

    plsc.subcore_barrier()

    def batch(b, carry):
        pltpu.sync_copy(
            src_hbm.at[pl.ds(c * NROW + s * AC + b * HB, HB)], src_v)
        pltpu.sync_copy(dst_hbm.at[pl.ds(s * AC + b * HB, HB)], dst_v)
        r0 = rows0.at[pl.ds(0, K)]
        r1 = rows1.at[pl.ds(0, K)]
        pltpu.async_copy(y_hbm.at[src_v.at[0]], r0, sem0)

        def ebody(t, carry2):
            j0 = 2 * t
            j1 = j0 + 1
            d1 = pltpu.async_copy(y_hbm.at[src_v.at[j1]], r1, sem1)
            pltpu.make_async_copy(y_hbm.at[src_v.at[j0]], r0, sem0).wait()
            pltpu.sync_copy(r0, acc.at[dst_v.at[j0]], add=True)

            @pl.when(j0 + 2 < HB)
            def _():
                pltpu.async_copy(y_hbm.at[src_v.at[j0 + 2]], r0, sem0)

            d1.wait()
            pltpu.sync_copy(r1, acc.at[dst_v.at[j1]], add=True)
            return carry2

        lax.fori_loop(0, HB // 2, ebody, 0)
        return carry

    lax.fori_loop(0, 2, batch, 0)

    plsc.subcore_barrier()

    def obody(j, carry):
        off = s * RPT + j * RB
        pltpu.sync_copy(acc.at[pl.ds(off, RB)], rows0)
        pltpu.sync_copy(rows0, o_hbm.at[pl.ds(base + off, RB)])
        return carry

    lax.fori_loop(0, RPT // RB, obody, 0)


_agg_kernel = pl.kernel(
    _agg_body,
    out_type=jax.ShapeDtypeStruct((2 * NNP, H), jnp.float32),
    mesh=_sc_mesh,
    scratch_types=[
        pltpu.VMEM((HB, K), jnp.int32),
        pltpu.VMEM((HB, K), jnp.int32),
        pltpu.VMEM((RB, H), jnp.float32),
        pltpu.VMEM((RB, H), jnp.float32),
        pltpu.VMEM_SHARED((NNP, H), jnp.float32),
        pltpu.SemaphoreType.DMA,
        pltpu.SemaphoreType.DMA,
    ],
)


def _dinv_of(degp_ref):
    deg = jnp.sum(degp_ref[...], axis=1) + 1.0
    return lax.rsqrt(deg)


def _tc_in_body(x_ref, w1_ref, degp_ref, y0_ref, y1_ref):
    dinv = _dinv_of(degp_ref)
    xw = jnp.dot(x_ref[...], w1_ref[...], preferred_element_type=jnp.float32)
    y = xw * dinv[:, None]
    y0_ref[...] = y[:, :H]
    y1_ref[...] = y[:, H:]


def _tc_mid_body(a0_ref, a1_ref, degp_ref, b1_ref, w2_ref,
                 h1_ref, y0_ref, y1_ref):
    dinv = _dinv_of(degp_ref)
    agg = jnp.concatenate([a0_ref[...], a1_ref[...]], axis=1)
    h1 = jnp.maximum(agg * dinv[:, None] + b1_ref[0, :], 0.0)
    h1_ref[...] = h1
    y2 = jnp.dot(h1, w2_ref[...], preferred_element_type=jnp.float32)
    y2 = y2 * dinv[:, None]
    y0_ref[...] = y2[:, :H]
    y1_ref[...] = y2[:, H:]


def _tc_out_body(a0_ref, a1_ref, degp_ref, b2_ref, h1_ref, wr_ref, br_ref,
                 out_ref):
    dinv = _dinv_of(degp_ref)
    agg = jnp.concatenate([a0_ref[...], a1_ref[...]], axis=1)
    h2 = jnp.maximum(agg * dinv[:, None] + b2_ref[0, :], 0.0)
    logits = (
        jnp.dot(h1_ref[...], wr_ref[:D, :], preferred_element_type=jnp.float32)
        + jnp.dot(h2, wr_ref[D:, :], preferred_element_type=jnp.float32)
        + br_ref[0, :]
    )
    m = jnp.max(logits, axis=1, keepdims=True)
    z = logits - m
    lse = jnp.log(jnp.sum(jnp.exp(z), axis=1, keepdims=True))
    out_ref[...] = z - lse


def _node_blk(shape_minor):
    return pl.BlockSpec((BN,) + shape_minor, lambda i: (i,) + (0,) * len(shape_minor))


def _slab_blk(half):
    return pl.BlockSpec((BN, H), lambda i, _h=half: (_h * GRID + i, 0))


_degp_blk = pl.BlockSpec((BN, 2), lambda i: (i, 0))


def _full_blk(shape):
    return pl.BlockSpec(shape, lambda i: (0,) * len(shape))


_tc_in = pl.pallas_call(
    _tc_in_body,
    grid=(GRID,),
    in_specs=[_node_blk((D,)), _full_blk((D, D)), _degp_blk],
    out_specs=[_node_blk((H,)), _node_blk((H,))],
    out_shape=(
        jax.ShapeDtypeStruct((NNP, H), jnp.float32),
        jax.ShapeDtypeStruct((NNP, H), jnp.float32),
    ),
)

_tc_mid = pl.pallas_call(
    _tc_mid_body,
    grid=(GRID,),
    in_specs=[_slab_blk(0), _slab_blk(1), _degp_blk,
              _full_blk((1, D)), _full_blk((D, D))],
    out_specs=[_node_blk((D,)), _node_blk((H,)), _node_blk((H,))],
    out_shape=(
        jax.ShapeDtypeStruct((N, D), jnp.float32),
        jax.ShapeDtypeStruct((NNP, H), jnp.float32),
        jax.ShapeDtypeStruct((NNP, H), jnp.float32),
    ),
)

_tc_out = pl.pallas_call(
    _tc_out_body,
    grid=(GRID,),
    in_specs=[_slab_blk(0), _slab_blk(1), _degp_blk,
              _full_blk((1, D)), _node_blk((D,)), _full_blk((2 * D, 40)),
              _full_blk((1, 40))],
    out_specs=_node_blk((40,)),
    out_shape=jax.ShapeDtypeStruct((N, 40), jnp.float32),
)


def kernel(x, edge_index, W1, b1, W2, b2, Wr, br):
    src2d = edge_index[0].astype(jnp.int32).reshape(NROW, K)
    dst2d = edge_index[1].astype(jnp.int32).reshape(NROW, K)
    gsrc2d = jnp.concatenate([src2d, src2d + NNP], axis=0)

    degp = _deg_kernel(dst2d).reshape(2, N).T

    y0, y1 = _tc_in(x, W1, degp)
    a = _agg_kernel(gsrc2d, dst2d, jnp.concatenate([y0, y1], axis=0))
    h1, z0, z1 = _tc_mid(a, a, degp, b1.reshape(1, D), W2)
    g = _agg_kernel(gsrc2d, dst2d, jnp.concatenate([z0, z1], axis=0))
    return _tc_out(g, g, degp, b2.reshape(1, D), h1, Wr, br.reshape(1, 40))
